# Initial kernel scaffold; baseline (speedup 1.0000x reference)
import functools

import jax
import jax.numpy as jnp
from jax.experimental import pallas as pl

N_NODES = 10000
N_EDGES = 320000
ROW_BLK = 1000


def _mm2_body(x_ref, wl_ref, wr_ref, xl_ref, xr_ref):
    x = x_ref[...]
    xl_ref[...] = jnp.dot(x, wl_ref[...], preferred_element_type=jnp.float32)
    xr_ref[...] = jnp.dot(x, wr_ref[...], preferred_element_type=jnp.float32)


def _mm2(x, wl, wr):
    n, d_in = x.shape
    d_out = wl.shape[1]
    grid = (n // ROW_BLK,)
    return pl.pallas_call(
        _mm2_body,
        grid=grid,
        in_specs=[
            pl.BlockSpec((ROW_BLK, d_in), lambda i: (i, 0)),
            pl.BlockSpec((d_in, d_out), lambda i: (0, 0)),
            pl.BlockSpec((d_in, d_out), lambda i: (0, 0)),
        ],
        out_specs=[
            pl.BlockSpec((ROW_BLK, d_out), lambda i: (i, 0)),
            pl.BlockSpec((ROW_BLK, d_out), lambda i: (i, 0)),
        ],
        out_shape=[
            jax.ShapeDtypeStruct((n, d_out), jnp.float32),
            jax.ShapeDtypeStruct((n, d_out), jnp.float32),
        ],
    )(x, wl, wr)


def _edge_pass(xl, xr, src, dst, ee, att):
    h = xl[src] + xr[dst] + ee
    h = jnp.where(h > 0, h, 0.2 * h)
    logits = h @ att
    w = jnp.exp(jnp.minimum(logits, 60.0))
    den = jax.ops.segment_sum(w, dst, num_segments=N_NODES)
    num = jax.ops.segment_sum(xl[src] * w[:, None], dst, num_segments=N_NODES)
    return num, den


def _norm_body(num_ref, den_ref, b_ref, o_ref):
    o = num_ref[...] / (den_ref[...] + 1e-16) + b_ref[...]
    o_ref[...] = jnp.where(o > 0, o, jnp.expm1(o))


def _norm_elu(num, den, b):
    n, d = num.shape
    return pl.pallas_call(
        _norm_body,
        grid=(n // ROW_BLK,),
        in_specs=[
            pl.BlockSpec((ROW_BLK, d), lambda i: (i, 0)),
            pl.BlockSpec((ROW_BLK, 1), lambda i: (i, 0)),
            pl.BlockSpec((1, d), lambda i: (0, 0)),
        ],
        out_specs=pl.BlockSpec((ROW_BLK, d), lambda i: (i, 0)),
        out_shape=jax.ShapeDtypeStruct((n, d), jnp.float32),
    )(num, den[:, None], b[None, :])


def kernel(x, edge_index, edge_attr, Wl1, Wr1, We1, att1, b1,
           Wl2, Wr2, We2, att2, b2):
    src = edge_index[0].astype(jnp.int32)
    dst = edge_index[1].astype(jnp.int32)

    xl1, xr1 = _mm2(x, Wl1, Wr1)
    ee1 = edge_attr * We1[0][None, :]
    num1, den1 = _edge_pass(xl1, xr1, src, dst, ee1, att1)
    h = _norm_elu(num1, den1, b1)

    xl2, xr2 = _mm2(h, Wl2, Wr2)
    ee2 = edge_attr * We2[0][None, :]
    num2, den2 = _edge_pass(xl2, xr2, src, dst, ee2, att2)
    return _norm_elu(num2, den2, b2)


# speedup vs baseline: 2.6133x; 2.6133x over previous
import functools

import jax
import jax.numpy as jnp
from jax.experimental import pallas as pl

N_NODES = 10000
N_EDGES = 320000
ROW_BLK = 1000


def _mm2_body(x_ref, wl_ref, wr_ref, xl_ref, xr_ref):
    x = x_ref[...]
    xl_ref[...] = jnp.dot(x, wl_ref[...], preferred_element_type=jnp.float32)
    xr_ref[...] = jnp.dot(x, wr_ref[...], preferred_element_type=jnp.float32)


def _mm2(x, wl, wr):
    n, d_in = x.shape
    d_out = wl.shape[1]
    grid = (n // ROW_BLK,)
    return pl.pallas_call(
        _mm2_body,
        grid=grid,
        in_specs=[
            pl.BlockSpec((ROW_BLK, d_in), lambda i: (i, 0)),
            pl.BlockSpec((d_in, d_out), lambda i: (0, 0)),
            pl.BlockSpec((d_in, d_out), lambda i: (0, 0)),
        ],
        out_specs=[
            pl.BlockSpec((ROW_BLK, d_out), lambda i: (i, 0)),
            pl.BlockSpec((ROW_BLK, d_out), lambda i: (i, 0)),
        ],
        out_shape=[
            jax.ShapeDtypeStruct((n, d_out), jnp.float32),
            jax.ShapeDtypeStruct((n, d_out), jnp.float32),
        ],
    )(x, wl, wr)


def _edge_pass(xl, xr, src, dst, ee, att):
    h = xl[src] + xr[dst] + ee
    h = jnp.where(h > 0, h, 0.2 * h)
    logits = h @ att
    w = jnp.exp(jnp.minimum(logits, 60.0))
    den = jax.ops.segment_sum(w, dst, num_segments=N_NODES)
    num = jax.ops.segment_sum(xl[src] * w[:, None], dst, num_segments=N_NODES)
    return num, den


def _norm_body(num_ref, den_ref, b_ref, o_ref):
    o = num_ref[...] / (den_ref[...] + 1e-16) + b_ref[...]
    o_ref[...] = jnp.where(o > 0, o, jnp.exp(jnp.minimum(o, 0.0)) - 1.0)


def _norm_elu(num, den, b):
    n, d = num.shape
    return pl.pallas_call(
        _norm_body,
        grid=(n // ROW_BLK,),
        in_specs=[
            pl.BlockSpec((ROW_BLK, d), lambda i: (i, 0)),
            pl.BlockSpec((ROW_BLK, 1), lambda i: (i, 0)),
            pl.BlockSpec((1, d), lambda i: (0, 0)),
        ],
        out_specs=pl.BlockSpec((ROW_BLK, d), lambda i: (i, 0)),
        out_shape=jax.ShapeDtypeStruct((n, d), jnp.float32),
    )(num, den[:, None], b[None, :])


def kernel(x, edge_index, edge_attr, Wl1, Wr1, We1, att1, b1,
           Wl2, Wr2, We2, att2, b2):
    src = edge_index[0].astype(jnp.int32)
    dst = edge_index[1].astype(jnp.int32)

    xl1, xr1 = _mm2(x, Wl1, Wr1)
    ee1 = edge_attr * We1[0][None, :]
    num1, den1 = _edge_pass(xl1, xr1, src, dst, ee1, att1)
    h = _norm_elu(num1, den1, b1)

    xl2, xr2 = _mm2(h, Wl2, Wr2)
    ee2 = edge_attr * We2[0][None, :]
    num2, den2 = _edge_pass(xl2, xr2, src, dst, ee2, att2)
    return _norm_elu(num2, den2, b2)


# speedup vs baseline: 7.8698x; 3.0114x over previous
import functools

import jax
import jax.numpy as jnp
from jax import lax
from jax.experimental import pallas as pl
from jax.experimental.pallas import tpu as pltpu
from jax.experimental.pallas import tpu_sc as plsc

N_NODES = 10000
N_EDGES = 320000
D = 64
ACC_W = 80
NC, NS = 2, 16
NW = NC * NS
EPW = N_EDGES // NW
CHUNK = 128
NCHUNK = (EPW + CHUNK - 1) // CHUNK
ROWS_PER_SUB = N_NODES // NS
_COPY_PLAN = [(0, 128), (128, 128), (256, 128), (384, 128), (512, 112)]
ROW_BLK = 1000


def _mm2_body(x_ref, wl_ref, wr_ref, xl_ref, xr_ref):
    x = x_ref[...]
    xl_ref[...] = jnp.dot(x, wl_ref[...], preferred_element_type=jnp.float32)
    xr_ref[...] = jnp.dot(x, wr_ref[...], preferred_element_type=jnp.float32)


def _mm2(x, wl, wr):
    n, d_in = x.shape
    d_out = wl.shape[1]
    return pl.pallas_call(
        _mm2_body,
        grid=(n // ROW_BLK,),
        in_specs=[
            pl.BlockSpec((ROW_BLK, d_in), lambda i: (i, 0)),
            pl.BlockSpec((d_in, d_out), lambda i: (0, 0)),
            pl.BlockSpec((d_in, d_out), lambda i: (0, 0)),
        ],
        out_specs=[
            pl.BlockSpec((ROW_BLK, d_out), lambda i: (i, 0)),
            pl.BlockSpec((ROW_BLK, d_out), lambda i: (i, 0)),
        ],
        out_shape=[
            jax.ShapeDtypeStruct((n, d_out), jnp.float32),
            jax.ShapeDtypeStruct((n, d_out), jnp.float32),
        ],
    )(x, wl, wr)


def _acc_to_act(acc, b):
    num = acc[0, :, :D] + acc[1, :, :D]
    den = acc[0, :, D:D + 1] + acc[1, :, D:D + 1]
    o = num / (den + 1e-16) + b
    return jnp.where(o > 0, o, jnp.exp(jnp.minimum(o, 0.0)) - 1.0)


def _nmm_body(acc_ref, b_ref, wl_ref, wr_ref, xl_ref, xr_ref):
    act = _acc_to_act(acc_ref[...], b_ref[...])
    xl_ref[...] = jnp.dot(act, wl_ref[...], preferred_element_type=jnp.float32)
    xr_ref[...] = jnp.dot(act, wr_ref[...], preferred_element_type=jnp.float32)


def _norm_mm2(acc, b, wl, wr):
    d_out = wl.shape[1]
    return pl.pallas_call(
        _nmm_body,
        grid=(N_NODES // ROW_BLK,),
        in_specs=[
            pl.BlockSpec((2, ROW_BLK, ACC_W), lambda i: (0, i, 0)),
            pl.BlockSpec((1, D), lambda i: (0, 0)),
            pl.BlockSpec((D, d_out), lambda i: (0, 0)),
            pl.BlockSpec((D, d_out), lambda i: (0, 0)),
        ],
        out_specs=[
            pl.BlockSpec((ROW_BLK, d_out), lambda i: (i, 0)),
            pl.BlockSpec((ROW_BLK, d_out), lambda i: (i, 0)),
        ],
        out_shape=[
            jax.ShapeDtypeStruct((N_NODES, d_out), jnp.float32),
            jax.ShapeDtypeStruct((N_NODES, d_out), jnp.float32),
        ],
    )(acc, b[None, :], wl, wr)


def _norm_body(acc_ref, b_ref, o_ref):
    o_ref[...] = _acc_to_act(acc_ref[...], b_ref[...])


def _norm_elu(acc, b):
    return pl.pallas_call(
        _norm_body,
        grid=(N_NODES // ROW_BLK,),
        in_specs=[
            pl.BlockSpec((2, ROW_BLK, ACC_W), lambda i: (0, i, 0)),
            pl.BlockSpec((1, D), lambda i: (0, 0)),
        ],
        out_specs=pl.BlockSpec((ROW_BLK, D), lambda i: (i, 0)),
        out_shape=jax.ShapeDtypeStruct((N_NODES, D), jnp.float32),
    )(acc, b[None, :])


_MESH = plsc.VectorSubcoreMesh(core_axis_name="c", subcore_axis_name="s")


@functools.partial(
    pl.kernel,
    out_type=jax.ShapeDtypeStruct((NC, N_NODES, ACC_W), jnp.float32),
    mesh=_MESH,
    scratch_types=[
        pltpu.VMEM((1, CHUNK), jnp.int32),
        pltpu.VMEM((1, CHUNK), jnp.int32),
        pltpu.VMEM((1, CHUNK), jnp.float32),
        pltpu.VMEM((CHUNK, D), jnp.float32),
        pltpu.VMEM((CHUNK, D), jnp.float32),
        pltpu.VMEM((CHUNK, ACC_W), jnp.float32),
        pltpu.VMEM((D,), jnp.float32),
        pltpu.VMEM((D,), jnp.float32),
        pltpu.VMEM((16, 16), jnp.float32),
        pltpu.VMEM_SHARED((N_NODES, ACC_W), jnp.float32),
        pltpu.SemaphoreType.DMA,
        pltpu.SemaphoreType.DMA,
    ],
    compiler_params=pltpu.CompilerParams(needs_layout_passes=False,
                                         use_tc_tiling_on_sc=False),
)
def _edge_kernel(xl_hbm, xr_hbm, srcp, dstp, eap, wev_hbm, attv_hbm, out_hbm,
                 src_v, dst_v, ea_v, buf_s, buf_d, msg, wev, attv, tbuf, acc,
                 sem_s, sem_d):
    cid = lax.axis_index("c")
    sid = lax.axis_index("s")
    wid = cid * NS + sid

    zeros16 = jnp.zeros((16,), jnp.float32)
    for r in range(CHUNK):
        for k in range(ACC_W // 16):
            msg[r, pl.ds(k * 16, 16)] = zeros16
    base = sid * 624
    for off, cnt in _COPY_PLAN:
        pltpu.sync_copy(msg.at[pl.ds(0, cnt)],
                        acc.at[pl.ds(base + off, cnt)])

    @pl.when(sid == NS - 1)
    def _():
        pltpu.sync_copy(msg.at[pl.ds(0, 16)], acc.at[pl.ds(9984, 16)])

    pltpu.sync_copy(wev_hbm, wev)
    pltpu.sync_copy(attv_hbm, attv)
    plsc.subcore_barrier()

    we_r = [wev[pl.ds(k * 16, 16)] for k in range(4)]
    at_r = [attv[pl.ds(k * 16, 16)] for k in range(4)]
    lane = lax.iota(jnp.int32, 16)
    col_d = jnp.full((16,), D, jnp.int32)

    def chunk_body(c, carry):
        ci = wid * NCHUNK + c
        pltpu.sync_copy(srcp.at[ci], src_v)
        pltpu.sync_copy(dstp.at[ci], dst_v)
        pltpu.sync_copy(eap.at[ci], ea_v)
        cp_s = pltpu.async_copy(xl_hbm.at[src_v.at[0]], buf_s, sem_s)
        cp_d = pltpu.async_copy(xr_hbm.at[dst_v.at[0]], buf_d, sem_d)
        cp_s.wait()
        cp_d.wait()
        for g in range(CHUNK // 16):
            ea16 = ea_v[0, pl.ds(g * 16, 16)]
            for j in range(16):
                row = g * 16 + j
                ea_j = ea16[j]
                dot = None
                for k in range(4):
                    h = (buf_s[row, pl.ds(k * 16, 16)]
                         + buf_d[row, pl.ds(k * 16, 16)]
                         + ea_j * we_r[k])
                    h = jnp.maximum(h, 0.2 * h)
                    t = h * at_r[k]
                    dot = t if dot is None else dot + t
                plsc.store_scatter(tbuf, [lane, jnp.full((16,), j, jnp.int32)],
                                   dot)
            lg = None
            for r in range(16):
                t = tbuf[r, pl.ds(0, 16)]
                lg = t if lg is None else lg + t
            ids = c * CHUNK + g * 16 + lane
            w16 = jnp.exp(jnp.minimum(lg, 60.0))
            w16 = jnp.where(ids < EPW, w16, 0.0)
            plsc.store_scatter(msg, [g * 16 + lane, col_d], w16)
            for j in range(16):
                row = g * 16 + j
                w_j = w16[j]
                for k in range(4):
                    msg[row, pl.ds(k * 16, 16)] = (
                        buf_s[row, pl.ds(k * 16, 16)] * w_j)
        pltpu.sync_copy(msg, acc.at[dst_v.at[0]], add=True)
        return carry

    lax.fori_loop(0, NCHUNK, chunk_body, 0)
    plsc.subcore_barrier()
    for off, cnt in _COPY_PLAN:
        pltpu.sync_copy(acc.at[pl.ds(base + off, cnt)],
                        out_hbm.at[cid, pl.ds(base + off, cnt)])

    @pl.when(sid == NS - 1)
    def _():
        pltpu.sync_copy(acc.at[pl.ds(9984, 16)],
                        out_hbm.at[cid, pl.ds(9984, 16)])


def _prep_edges(a):
    a = a.reshape(NW, EPW)
    a = jnp.pad(a, ((0, 0), (0, NCHUNK * CHUNK - EPW)))
    return a.reshape(NW * NCHUNK, 1, CHUNK)


def kernel(x, edge_index, edge_attr, Wl1, Wr1, We1, att1, b1,
           Wl2, Wr2, We2, att2, b2):
    srcp = _prep_edges(edge_index[0].astype(jnp.int32))
    dstp = _prep_edges(edge_index[1].astype(jnp.int32))
    eap = _prep_edges(edge_attr[:, 0])

    xl1, xr1 = _mm2(x, Wl1, Wr1)
    acc1 = _edge_kernel(xl1, xr1, srcp, dstp, eap, We1[0], att1)
    xl2, xr2 = _norm_mm2(acc1, b1, Wl2, Wr2)
    acc2 = _edge_kernel(xl2, xr2, srcp, dstp, eap, We2[0], att2)
    return _norm_elu(acc2, b2)
